# baseline (speedup 1.0000x reference)
import functools

import jax
import jax.numpy as jnp
from jax import lax
from jax.experimental import pallas as pl
from jax.experimental.pallas import tpu as pltpu
from jax.experimental.pallas import tpu_sc as plsc

N = 50000
E = 800000
D = 64
C = 64

NSUB = 16
NP = 50176
HNP = NP // 2
ROWS_PER_SUB = NP // NSUB
ZCH = 392
ASUM_PER_SUB = HNP // NSUB
CHUNK = 128
NCHUNKS = E // CHUNK
ITERS_PAIRED = 392

HALF = D // 2

BR = 400


def _k0_body(x_ref, w_ref, ati_ref, atj_ref, h0_ref, h1_ref, ai_ref, aj_ref):
    h = jnp.dot(x_ref[...], w_ref[...], preferred_element_type=jnp.float32)
    h0_ref[...] = h[:, :HALF]
    h1_ref[...] = h[:, HALF:]
    ai_ref[...] = jnp.dot(h, ati_ref[...], preferred_element_type=jnp.float32)
    aj_ref[...] = jnp.dot(h, atj_ref[...], preferred_element_type=jnp.float32)


def _run_k0(x_p, W, ati, atj):
    grid = (N // BR,)
    return pl.pallas_call(
        _k0_body,
        grid=grid,
        in_specs=[
            pl.BlockSpec((BR, D), lambda i: (i, 0)),
            pl.BlockSpec((D, D), lambda i: (0, 0)),
            pl.BlockSpec((D, 1), lambda i: (0, 0)),
            pl.BlockSpec((D, 1), lambda i: (0, 0)),
        ],
        out_specs=[
            pl.BlockSpec((BR, HALF), lambda i: (i, 0)),
            pl.BlockSpec((BR, HALF), lambda i: (i, 0)),
            pl.BlockSpec((BR, 1), lambda i: (i, 0)),
            pl.BlockSpec((BR, 1), lambda i: (i, 0)),
        ],
        out_shape=[
            jax.ShapeDtypeStruct((N, HALF), jnp.float32),
            jax.ShapeDtypeStruct((N, HALF), jnp.float32),
            jax.ShapeDtypeStruct((N, 1), jnp.float32),
            jax.ShapeDtypeStruct((N, 1), jnp.float32),
        ],
    )(x_p, W, ati, atj)


def _k1_body(src_hbm, dst_hbm, h0_hbm, h1_hbm, ai_hbm, aj_hbm,
             aggr0_out, aggr1_out, asum_out,
             src_v0, dst_v0, lidx_v0, ai_v0, aj_v0, w_v0, rows_v0,
             src_v1, dst_v1, lidx_v1, ai_v1, aj_v1, w_v1, rows_v1,
             zbuf, zbuf1d,
             aggr_sh, asum_sh, sem_g0, sem_g1, sem_s0, sem_s1):
    cid = lax.axis_index("c")
    sid = lax.axis_index("s")
    sets = [
        (src_v0, dst_v0, lidx_v0, ai_v0, aj_v0, w_v0, rows_v0, sem_g0, sem_s0),
        (src_v1, dst_v1, lidx_v1, ai_v1, aj_v1, w_v1, rows_v1, sem_g1, sem_s1),
    ]

    def _zrow(i, carry):
        zbuf[i, pl.ds(0, 16)] = jnp.zeros((16,), jnp.float32)
        zbuf[i, pl.ds(16, 16)] = jnp.zeros((16,), jnp.float32)
        return carry

    lax.fori_loop(0, ZCH, _zrow, 0)

    def _zrow1(i, carry):
        zbuf1d[pl.ds(i * 16, 16)] = jnp.zeros((16,), jnp.float32)
        return carry

    lax.fori_loop(0, ASUM_PER_SUB // 16, _zrow1, 0)

    base = sid * ROWS_PER_SUB
    for k in range(ROWS_PER_SUB // ZCH):
        pltpu.sync_copy(zbuf, aggr_sh.at[pl.ds(base + k * ZCH, ZCH)])
    abase = sid * ASUM_PER_SUB
    pltpu.sync_copy(zbuf1d, asum_sh.at[pl.ds(abase, ASUM_PER_SUB)])

    @pl.when(sid == 0)
    def _():
        pltpu.sync_copy(zbuf1d.at[pl.ds(0, 16)], asum_sh.at[pl.ds(HNP, 16)])

    plsc.subcore_barrier()

    lo = cid * HNP

    def _issue(g, b):
        src_b, dst_b, _, ai_b, aj_b, _, rows_b, sem_gb, _ = sets[b]
        chunk_id = g * NSUB + sid
        off = jnp.where(chunk_id < NCHUNKS, chunk_id, 0) * CHUNK
        pltpu.sync_copy(src_hbm.at[pl.ds(off, CHUNK)], src_b)
        pltpu.sync_copy(dst_hbm.at[pl.ds(off, CHUNK)], dst_b)
        pltpu.async_copy(ai_hbm.at[dst_b], ai_b, sem_gb)
        pltpu.async_copy(aj_hbm.at[src_b], aj_b, sem_gb)

        @pl.when(cid == 0)
        def _():
            pltpu.async_copy(h0_hbm.at[src_b], rows_b, sem_gb)

        @pl.when(cid == 1)
        def _():
            pltpu.async_copy(h1_hbm.at[src_b], rows_b, sem_gb)

    def _wait_gathers(b):
        src_b, dst_b, _, ai_b, aj_b, _, rows_b, sem_gb, _ = sets[b]
        pltpu.make_async_copy(ai_hbm.at[dst_b], ai_b, sem_gb).wait()
        pltpu.make_async_copy(aj_hbm.at[src_b], aj_b, sem_gb).wait()

        @pl.when(cid == 0)
        def _():
            pltpu.make_async_copy(h0_hbm.at[src_b], rows_b, sem_gb).wait()

        @pl.when(cid == 1)
        def _():
            pltpu.make_async_copy(h1_hbm.at[src_b], rows_b, sem_gb).wait()

    def _compute(g, b):
        src_b, dst_b, lidx_b, ai_b, aj_b, w_b, rows_b, _, _ = sets[b]
        chunk_id = g * NSUB + sid
        vmask = (chunk_id < NCHUNKS).astype(jnp.float32)

        def _grp(i, carry2):
            sl = pl.ds(i * 16, 16)
            a = ai_b[sl] + aj_b[sl]
            a = jnp.maximum(a, 0.2 * a)
            w = jnp.exp(a) * vmask
            keep = src_b[sl] != dst_b[sl]
            w = jnp.where(keep, w, jnp.zeros((16,), jnp.float32))
            w_b[sl] = w
            loc = dst_b[sl] - lo
            inr = (loc >= 0) & (loc < HNP)
            lidx_b[sl] = jnp.where(inr, loc, HNP)
            for j in range(16):
                r = i * 16 + j
                wr = w[j]
                rows_b[r, pl.ds(0, 16)] = rows_b[r, pl.ds(0, 16)] * wr
                rows_b[r, pl.ds(16, 16)] = rows_b[r, pl.ds(16, 16)] * wr
            return carry2

        lax.fori_loop(0, CHUNK // 16, _grp, 0)

    def _issue_scatter(b):
        _, dst_b, lidx_b, _, _, w_b, rows_b, _, sem_sb = sets[b]
        pltpu.async_copy(rows_b, aggr_sh.at[dst_b], sem_sb, add=True)
        pltpu.async_copy(w_b, asum_sh.at[lidx_b], sem_sb, add=True)

    def _wait_scatter(b):
        _, dst_b, lidx_b, _, _, w_b, rows_b, _, sem_sb = sets[b]
        pltpu.make_async_copy(rows_b, aggr_sh.at[dst_b], sem_sb).wait()
        pltpu.make_async_copy(w_b, asum_sh.at[lidx_b], sem_sb).wait()

    _issue(0, 0)

    def _pair(go, carry):
        for b in range(2):
            g = go * 2 + b
            q = 1 - b
            _wait_gathers(b)
            _compute(g, b)

            @pl.when(g > 0)
            def _():
                _wait_scatter(q)

            _issue(g + 1, q)
            _issue_scatter(b)
        return carry

    lax.fori_loop(0, ITERS_PAIRED // 2, _pair, 0)
    _wait_gathers(0)
    _wait_scatter(1)
    plsc.subcore_barrier()

    for k in range(ROWS_PER_SUB // ZCH):
        r0 = base + k * ZCH
        pltpu.sync_copy(aggr_sh.at[pl.ds(r0, ZCH)], zbuf)

        @pl.when(cid == 0)
        def _():
            pltpu.sync_copy(zbuf, aggr0_out.at[pl.ds(r0, ZCH)])

        @pl.when(cid == 1)
        def _():
            pltpu.sync_copy(zbuf, aggr1_out.at[pl.ds(r0, ZCH)])

    pltpu.sync_copy(asum_sh.at[pl.ds(abase, ASUM_PER_SUB)], zbuf1d)
    pltpu.sync_copy(zbuf1d, asum_out.at[pl.ds(lo + abase, ASUM_PER_SUB)])


def _run_k1(src, dst, h0, h1, ai, aj):
    mesh = plsc.VectorSubcoreMesh(core_axis_name="c", subcore_axis_name="s")
    k1 = functools.partial(
        pl.kernel,
        mesh=mesh,
        compiler_params=pltpu.CompilerParams(use_tc_tiling_on_sc=False),
        out_type=[
            jax.ShapeDtypeStruct((NP, HALF), jnp.float32),
            jax.ShapeDtypeStruct((NP, HALF), jnp.float32),
            jax.ShapeDtypeStruct((NP,), jnp.float32),
        ],
        scratch_types=(
            [
                pltpu.VMEM((CHUNK,), jnp.int32),
                pltpu.VMEM((CHUNK,), jnp.int32),
                pltpu.VMEM((CHUNK,), jnp.int32),
                pltpu.VMEM((CHUNK,), jnp.float32),
                pltpu.VMEM((CHUNK,), jnp.float32),
                pltpu.VMEM((CHUNK,), jnp.float32),
                pltpu.VMEM((CHUNK, HALF), jnp.float32),
            ] * 2
            + [
                pltpu.VMEM((ZCH, HALF), jnp.float32),
                pltpu.VMEM((ASUM_PER_SUB,), jnp.float32),
                pltpu.VMEM_SHARED((NP, HALF), jnp.float32),
                pltpu.VMEM_SHARED((HNP + 16,), jnp.float32),
                pltpu.SemaphoreType.DMA,
                pltpu.SemaphoreType.DMA,
                pltpu.SemaphoreType.DMA,
                pltpu.SemaphoreType.DMA,
            ]
        ),
    )(_k1_body)
    return k1(src, dst, h0, h1, ai, aj)


def _k2_body(aggr0_ref, aggr1_ref, asum_ref, h0_ref, h1_ref, ai_ref, aj_ref,
             b_ref, out_ref):
    z = ai_ref[...] + aj_ref[...]
    w = jnp.exp(jnp.maximum(z, 0.2 * z))
    den = asum_ref[...] + w + 1e-16
    n0 = (aggr0_ref[...] + w * h0_ref[...]) / den
    n1 = (aggr1_ref[...] + w * h1_ref[...]) / den
    o = jnp.concatenate([n0, n1], axis=1) + b_ref[...]
    nrm = jnp.sqrt(jnp.sum(o * o, axis=1, keepdims=True))
    out_ref[...] = o / jnp.maximum(nrm, 1e-12)


def _run_k2(aggr0, aggr1, asum, h0, h1, ai, aj, b):
    grid = (N // BR,)
    return pl.pallas_call(
        _k2_body,
        grid=grid,
        in_specs=[
            pl.BlockSpec((BR, HALF), lambda i: (i, 0)),
            pl.BlockSpec((BR, HALF), lambda i: (i, 0)),
            pl.BlockSpec((BR, 1), lambda i: (i, 0)),
            pl.BlockSpec((BR, HALF), lambda i: (i, 0)),
            pl.BlockSpec((BR, HALF), lambda i: (i, 0)),
            pl.BlockSpec((BR, 1), lambda i: (i, 0)),
            pl.BlockSpec((BR, 1), lambda i: (i, 0)),
            pl.BlockSpec((1, D), lambda i: (0, 0)),
        ],
        out_specs=pl.BlockSpec((BR, D), lambda i: (i, 0)),
        out_shape=jax.ShapeDtypeStruct((N, D), jnp.float32),
    )(aggr0, aggr1, asum, h0, h1, ai, aj, b)


def kernel(x, edge_index, W, att, b):
    ati = att[0, 0, :C].reshape(D, 1)
    atj = att[0, 0, C:].reshape(D, 1)

    h0, h1, ai, aj = _run_k0(x, W, ati, atj)
    ai1 = ai.reshape(N)
    aj1 = aj.reshape(N)

    src = edge_index[0]
    dst = edge_index[1]

    aggr0, aggr1, asum = _run_k1(src, dst, h0, h1, ai1, aj1)

    return _run_k2(aggr0, aggr1, asum.reshape(NP, 1), h0, h1, ai, aj,
                   b.reshape(1, D))


# speedup vs baseline: 1.6513x; 1.6513x over previous
import functools

import jax
import jax.numpy as jnp
from jax import lax
from jax.experimental import pallas as pl
from jax.experimental.pallas import tpu as pltpu
from jax.experimental.pallas import tpu_sc as plsc

N = 50000
E = 800000
D = 64
C = 64

NSUB = 16
NP = 50176
HNP = NP // 2
ROWS_PER_SUB = NP // NSUB
ZCH = 112
ASUM_PER_SUB = HNP // NSUB
CHUNK = 128
SUBS = 2
BLK = CHUNK * SUBS
NBLKS = E // BLK
ITERS_Q = 196

HALF = D // 2

BR = 2000


def _k0_body(x_ref, w_ref, ati_ref, atj_ref, h0_ref, h1_ref, ai_ref, aj_ref):
    h = jnp.dot(x_ref[...], w_ref[...], preferred_element_type=jnp.float32)
    h0_ref[...] = h[:, :HALF]
    h1_ref[...] = h[:, HALF:]
    ai_ref[...] = jnp.dot(h, ati_ref[...], preferred_element_type=jnp.float32)
    aj_ref[...] = jnp.dot(h, atj_ref[...], preferred_element_type=jnp.float32)


def _run_k0(x_p, W, ati, atj):
    grid = (N // BR,)
    return pl.pallas_call(
        _k0_body,
        grid=grid,
        in_specs=[
            pl.BlockSpec((BR, D), lambda i: (i, 0)),
            pl.BlockSpec((D, D), lambda i: (0, 0)),
            pl.BlockSpec((D, 1), lambda i: (0, 0)),
            pl.BlockSpec((D, 1), lambda i: (0, 0)),
        ],
        out_specs=[
            pl.BlockSpec((BR, HALF), lambda i: (i, 0)),
            pl.BlockSpec((BR, HALF), lambda i: (i, 0)),
            pl.BlockSpec((BR, 1), lambda i: (i, 0)),
            pl.BlockSpec((BR, 1), lambda i: (i, 0)),
        ],
        out_shape=[
            jax.ShapeDtypeStruct((N, HALF), jnp.float32),
            jax.ShapeDtypeStruct((N, HALF), jnp.float32),
            jax.ShapeDtypeStruct((N, 1), jnp.float32),
            jax.ShapeDtypeStruct((N, 1), jnp.float32),
        ],
    )(x_p, W, ati, atj)


def _k1_body(src2_hbm, dst2_hbm, h0_hbm, h1_hbm, ai_hbm, aj_hbm,
             aggr0_out, aggr1_out, asum_out,
             ai2_0, aj2_0, lidx2_0, w2_0, rows0_0, rows1_0,
             ai2_1, aj2_1, lidx2_1, w2_1, rows0_1, rows1_1,
             srcs, dsts, zbuf, zbuf1d,
             aggr_sh, asum_sh,
             sem_i0, sem_i1, sem_i2, sem_i3,
             sem_g0, sem_g1, sem_s0, sem_s1):
    cid = lax.axis_index("c")
    sid = lax.axis_index("s")
    dsets = [
        (ai2_0, aj2_0, lidx2_0, w2_0, (rows0_0, rows1_0), sem_g0, sem_s0),
        (ai2_1, aj2_1, lidx2_1, w2_1, (rows0_1, rows1_1), sem_g1, sem_s1),
    ]
    semi = [sem_i0, sem_i1, sem_i2, sem_i3]

    def _zrow(i, carry):
        zbuf[i, pl.ds(0, 16)] = jnp.zeros((16,), jnp.float32)
        zbuf[i, pl.ds(16, 16)] = jnp.zeros((16,), jnp.float32)
        return carry

    lax.fori_loop(0, ZCH, _zrow, 0)

    def _zrow1(i, carry):
        zbuf1d[pl.ds(i * 16, 16)] = jnp.zeros((16,), jnp.float32)
        return carry

    lax.fori_loop(0, ASUM_PER_SUB // 16, _zrow1, 0)

    base = sid * ROWS_PER_SUB
    for k in range(ROWS_PER_SUB // ZCH):
        pltpu.sync_copy(zbuf, aggr_sh.at[pl.ds(base + k * ZCH, ZCH)])
    abase = sid * ASUM_PER_SUB
    pltpu.sync_copy(zbuf1d, asum_sh.at[pl.ds(abase, ASUM_PER_SUB)])

    @pl.when(sid == 0)
    def _():
        pltpu.sync_copy(zbuf1d.at[pl.ds(0, 16)], asum_sh.at[pl.ds(HNP, 16)])

    plsc.subcore_barrier()

    lo = cid * HNP

    def _issue_idx(g, slot):
        blk = g * NSUB + sid
        row0 = jnp.where(blk < NBLKS, blk, 0) * SUBS
        pltpu.async_copy(src2_hbm.at[pl.ds(row0, SUBS)],
                         srcs.at[pl.ds(2 * slot, SUBS)], semi[slot])
        pltpu.async_copy(dst2_hbm.at[pl.ds(row0, SUBS)],
                         dsts.at[pl.ds(2 * slot, SUBS)], semi[slot])

    def _wait_idx(g, slot):
        blk = g * NSUB + sid
        row0 = jnp.where(blk < NBLKS, blk, 0) * SUBS
        pltpu.make_async_copy(src2_hbm.at[pl.ds(row0, SUBS)],
                              srcs.at[pl.ds(2 * slot, SUBS)],
                              semi[slot]).wait()
        pltpu.make_async_copy(dst2_hbm.at[pl.ds(row0, SUBS)],
                              dsts.at[pl.ds(2 * slot, SUBS)],
                              semi[slot]).wait()

    def _issue_gathers(slot, b):
        ai_b, aj_b, _, _, rows_b, sem_gb, _ = dsets[b]
        for k in range(SUBS):
            ir = 2 * slot + k
            pltpu.async_copy(ai_hbm.at[dsts.at[ir]], ai_b.at[k], sem_gb)
            pltpu.async_copy(aj_hbm.at[srcs.at[ir]], aj_b.at[k], sem_gb)

            @pl.when(cid == 0)
            def _():
                pltpu.async_copy(h0_hbm.at[srcs.at[ir]], rows_b[k], sem_gb)

            @pl.when(cid == 1)
            def _():
                pltpu.async_copy(h1_hbm.at[srcs.at[ir]], rows_b[k], sem_gb)

    def _wait_gathers(slot, b):
        ai_b, aj_b, _, _, rows_b, sem_gb, _ = dsets[b]
        for k in range(SUBS):
            ir = 2 * slot + k
            pltpu.make_async_copy(ai_hbm.at[dsts.at[ir]], ai_b.at[k],
                                  sem_gb).wait()
            pltpu.make_async_copy(aj_hbm.at[srcs.at[ir]], aj_b.at[k],
                                  sem_gb).wait()

            @pl.when(cid == 0)
            def _():
                pltpu.make_async_copy(h0_hbm.at[srcs.at[ir]], rows_b[k],
                                      sem_gb).wait()

            @pl.when(cid == 1)
            def _():
                pltpu.make_async_copy(h1_hbm.at[srcs.at[ir]], rows_b[k],
                                      sem_gb).wait()

    def _compute(g, slot, b):
        ai_b, aj_b, lidx_b, w_b, rows_b, _, _ = dsets[b]
        vmask = (g * NSUB + sid < NBLKS).astype(jnp.float32)
        for k in range(SUBS):
            ir = 2 * slot + k
            rows_k = rows_b[k]

            def _grp(i, carry2):
                sl = pl.ds(i * 16, 16)
                a = ai_b[k, sl] + aj_b[k, sl]
                a = jnp.maximum(a, 0.2 * a)
                w = jnp.exp(a) * vmask
                keep = srcs[ir, sl] != dsts[ir, sl]
                w = jnp.where(keep, w, jnp.zeros((16,), jnp.float32))
                w_b[k, sl] = w
                loc = dsts[ir, sl] - lo
                inr = (loc >= 0) & (loc < HNP)
                lidx_b[k, sl] = jnp.where(inr, loc, HNP)
                for j in range(16):
                    r = i * 16 + j
                    wr = w[j]
                    rows_k[r, pl.ds(0, 16)] = rows_k[r, pl.ds(0, 16)] * wr
                    rows_k[r, pl.ds(16, 16)] = rows_k[r, pl.ds(16, 16)] * wr
                return carry2

            lax.fori_loop(0, CHUNK // 16, _grp, 0)

    def _issue_scatter(slot, b):
        _, _, lidx_b, w_b, rows_b, _, sem_sb = dsets[b]
        for k in range(SUBS):
            ir = 2 * slot + k
            pltpu.async_copy(rows_b[k], aggr_sh.at[dsts.at[ir]], sem_sb,
                             add=True)
            pltpu.async_copy(w_b.at[k], asum_sh.at[lidx_b.at[k]], sem_sb,
                             add=True)

    def _wait_scatter(slot, b):
        _, _, lidx_b, w_b, rows_b, _, sem_sb = dsets[b]
        for k in range(SUBS):
            ir = 2 * slot + k
            pltpu.make_async_copy(rows_b[k], aggr_sh.at[dsts.at[ir]],
                                  sem_sb).wait()
            pltpu.make_async_copy(w_b.at[k], asum_sh.at[lidx_b.at[k]],
                                  sem_sb).wait()

    _issue_idx(0, 0)
    _issue_idx(1, 1)
    _issue_idx(2, 2)
    _wait_idx(0, 0)
    _issue_gathers(0, 0)

    def _quad(qi, carry):
        for u in range(4):
            g = qi * 4 + u
            slot = u
            b = u % 2
            q = 1 - b
            nslot = (u + 1) % 4
            pslot = (u + 3) % 4
            _wait_gathers(slot, b)
            _compute(g, slot, b)

            @pl.when(g > 0)
            def _():
                _wait_scatter(pslot, q)

            _wait_idx(g + 1, nslot)
            _issue_gathers(nslot, q)
            _issue_idx(g + 3, pslot)
            _issue_scatter(slot, b)
        return carry

    lax.fori_loop(0, ITERS_Q // 4, _quad, 0)
    _wait_gathers(0, 0)
    _wait_scatter(3, 1)
    _wait_idx(ITERS_Q + 1, 1)
    _wait_idx(ITERS_Q + 2, 2)
    plsc.subcore_barrier()

    for k in range(ROWS_PER_SUB // ZCH):
        r0 = base + k * ZCH
        pltpu.sync_copy(aggr_sh.at[pl.ds(r0, ZCH)], zbuf)

        @pl.when(cid == 0)
        def _():
            pltpu.sync_copy(zbuf, aggr0_out.at[pl.ds(r0, ZCH)])

        @pl.when(cid == 1)
        def _():
            pltpu.sync_copy(zbuf, aggr1_out.at[pl.ds(r0, ZCH)])

    pltpu.sync_copy(asum_sh.at[pl.ds(abase, ASUM_PER_SUB)], zbuf1d)
    pltpu.sync_copy(zbuf1d, asum_out.at[pl.ds(lo + abase, ASUM_PER_SUB)])


def _run_k1(src2, dst2, h0, h1, ai, aj):
    mesh = plsc.VectorSubcoreMesh(core_axis_name="c", subcore_axis_name="s")
    k1 = functools.partial(
        pl.kernel,
        mesh=mesh,
        compiler_params=pltpu.CompilerParams(use_tc_tiling_on_sc=False),
        out_type=[
            jax.ShapeDtypeStruct((NP, HALF), jnp.float32),
            jax.ShapeDtypeStruct((NP, HALF), jnp.float32),
            jax.ShapeDtypeStruct((NP,), jnp.float32),
        ],
        scratch_types=(
            [
                pltpu.VMEM((SUBS, CHUNK), jnp.float32),
                pltpu.VMEM((SUBS, CHUNK), jnp.float32),
                pltpu.VMEM((SUBS, CHUNK), jnp.int32),
                pltpu.VMEM((SUBS, CHUNK), jnp.float32),
                pltpu.VMEM((CHUNK, HALF), jnp.float32),
                pltpu.VMEM((CHUNK, HALF), jnp.float32),
            ] * 2
            + [
                pltpu.VMEM((4 * SUBS, CHUNK), jnp.int32),
                pltpu.VMEM((4 * SUBS, CHUNK), jnp.int32),
                pltpu.VMEM((ZCH, HALF), jnp.float32),
                pltpu.VMEM((ASUM_PER_SUB,), jnp.float32),
                pltpu.VMEM_SHARED((NP, HALF), jnp.float32),
                pltpu.VMEM_SHARED((HNP + 16,), jnp.float32),
                pltpu.SemaphoreType.DMA,
                pltpu.SemaphoreType.DMA,
                pltpu.SemaphoreType.DMA,
                pltpu.SemaphoreType.DMA,
                pltpu.SemaphoreType.DMA,
                pltpu.SemaphoreType.DMA,
                pltpu.SemaphoreType.DMA,
                pltpu.SemaphoreType.DMA,
            ]
        ),
    )(_k1_body)
    return k1(src2, dst2, h0, h1, ai, aj)


def _k2_body(aggr0_ref, aggr1_ref, asum_ref, h0_ref, h1_ref, ai_ref, aj_ref,
             b_ref, out_ref):
    z = ai_ref[...] + aj_ref[...]
    w = jnp.exp(jnp.maximum(z, 0.2 * z))
    den = asum_ref[...] + w + 1e-16
    n0 = (aggr0_ref[...] + w * h0_ref[...]) / den
    n1 = (aggr1_ref[...] + w * h1_ref[...]) / den
    o = jnp.concatenate([n0, n1], axis=1) + b_ref[...]
    nrm = jnp.sqrt(jnp.sum(o * o, axis=1, keepdims=True))
    out_ref[...] = o / jnp.maximum(nrm, 1e-12)


def _run_k2(aggr0, aggr1, asum, h0, h1, ai, aj, b):
    grid = (N // BR,)
    return pl.pallas_call(
        _k2_body,
        grid=grid,
        in_specs=[
            pl.BlockSpec((BR, HALF), lambda i: (i, 0)),
            pl.BlockSpec((BR, HALF), lambda i: (i, 0)),
            pl.BlockSpec((BR, 1), lambda i: (i, 0)),
            pl.BlockSpec((BR, HALF), lambda i: (i, 0)),
            pl.BlockSpec((BR, HALF), lambda i: (i, 0)),
            pl.BlockSpec((BR, 1), lambda i: (i, 0)),
            pl.BlockSpec((BR, 1), lambda i: (i, 0)),
            pl.BlockSpec((1, D), lambda i: (0, 0)),
        ],
        out_specs=pl.BlockSpec((BR, D), lambda i: (i, 0)),
        out_shape=jax.ShapeDtypeStruct((N, D), jnp.float32),
    )(aggr0, aggr1, asum, h0, h1, ai, aj, b)


def kernel(x, edge_index, W, att, b):
    ati = att[0, 0, :C].reshape(D, 1)
    atj = att[0, 0, C:].reshape(D, 1)

    h0, h1, ai, aj = _run_k0(x, W, ati, atj)
    ai1 = ai.reshape(N)
    aj1 = aj.reshape(N)

    src2 = edge_index[0].reshape(E // CHUNK, CHUNK)
    dst2 = edge_index[1].reshape(E // CHUNK, CHUNK)

    aggr0, aggr1, asum = _run_k1(src2, dst2, h0, h1, ai1, aj1)

    return _run_k2(aggr0, aggr1, asum.reshape(NP, 1), h0, h1, ai, aj,
                   b.reshape(1, D))
